# baseline (speedup 1.0000x reference)
import functools

import jax
import jax.numpy as jnp
from jax import lax
from jax.experimental import pallas as pl
from jax.experimental.pallas import tpu as pltpu
from jax.experimental.pallas import tpu_sc as plsc

NCORE = 2
NSUB = 16
NWORK = NCORE * NSUB


def _mesh():
    return plsc.VectorSubcoreMesh(core_axis_name="c", subcore_axis_name="s")


def _make_deg(NP, E):
    EW = E // NWORK
    K = 2000
    G = EW // K
    RPS = NP // NSUB

    @functools.partial(
        pl.kernel,
        out_type=(jax.ShapeDtypeStruct((NP,), jnp.float32),
                  jax.ShapeDtypeStruct((NP,), jnp.float32)),
        mesh=_mesh(),
        scratch_types=[
            pltpu.VMEM((K,), jnp.int32),
            pltpu.VMEM((K,), jnp.float32),
            pltpu.VMEM_SHARED((NP,), jnp.float32),
        ],
    )
    def deg_kernel(dst_hbm, out0_hbm, out1_hbm, idx_v, ones_v, acc):
        c = lax.axis_index("c")
        s = lax.axis_index("s")
        wid = c * NSUB + s
        one16 = jnp.ones((16,), jnp.float32)
        z16 = jnp.zeros((16,), jnp.float32)

        def fill(i, carry):
            ones_v[pl.ds(i * 16, 16)] = jnp.where(i * 16 < RPS, z16, one16)
            return carry

        lax.fori_loop(0, K // 16, fill, 0)
        pltpu.sync_copy(ones_v.at[pl.ds(0, RPS)], acc.at[pl.ds(s * RPS, RPS)])
        plsc.subcore_barrier()

        def refill(i, carry):
            ones_v[pl.ds(i * 16, 16)] = one16
            return carry

        lax.fori_loop(0, RPS // 16, refill, 0)

        base = wid * EW

        def body(g, carry):
            pltpu.sync_copy(dst_hbm.at[pl.ds(base + g * K, K)], idx_v)
            pltpu.sync_copy(ones_v, acc.at[idx_v], add=True)
            return carry

        lax.fori_loop(0, G, body, 0)
        plsc.subcore_barrier()

        @pl.when(c == 0)
        def _():
            pltpu.sync_copy(acc.at[pl.ds(s * RPS, RPS)],
                            out0_hbm.at[pl.ds(s * RPS, RPS)])

        @pl.when(c == 1)
        def _():
            pltpu.sync_copy(acc.at[pl.ds(s * RPS, RPS)],
                            out1_hbm.at[pl.ds(s * RPS, RPS)])

    return deg_kernel


def _make_edge(NP, E, D, K, tc_tiling=True):
    EW = E // NWORK
    G = EW // K
    T = EW - G * K
    GS = ((G - 2) // 4) * 4
    RPS = NP // NSUB
    RC = RPS // 5
    DW = D // 16

    @functools.partial(
        pl.kernel,
        out_type=jax.ShapeDtypeStruct((NCORE, NP, D), jnp.float32),
        mesh=_mesh(),
        compiler_params=None if tc_tiling else pltpu.CompilerParams(
            use_tc_tiling_on_sc=False),
        scratch_types=[
            pltpu.VMEM((2, K), jnp.int32),
            pltpu.VMEM((2, K), jnp.int32),
            pltpu.VMEM((2, K), jnp.int32),
            pltpu.VMEM((2, K), jnp.int32),
            pltpu.VMEM((K, D), jnp.float32),
            pltpu.VMEM((K, D), jnp.float32),
            pltpu.SemaphoreType.DMA,
            pltpu.SemaphoreType.DMA,
            pltpu.SemaphoreType.DMA,
            pltpu.SemaphoreType.DMA,
            pltpu.SemaphoreType.DMA,
            pltpu.SemaphoreType.DMA,
            pltpu.VMEM((T if T > 0 else 16,), jnp.int32),
            pltpu.VMEM((T if T > 0 else 16,), jnp.int32),
            pltpu.VMEM_SHARED((NP, D), jnp.float32),
        ],
    )
    def edge_kernel(e3_hbm, tails_hbm, xs_hbm, out_hbm, eb0, eb1, eb2, eb3,
                    rows0, rows1, semg0, semg1, semi0, semi1, semi2, semi3,
                    tsrc, tdst, acc):
        c = lax.axis_index("c")
        s = lax.axis_index("s")
        wid = c * NSUB + s
        z16 = jnp.zeros((16,), jnp.float32)
        ebuf = (eb0, eb1, eb2, eb3)
        semi = (semi0, semi1, semi2, semi3)
        rows = (rows0, rows1)
        semg = (semg0, semg1)

        cbase = wid * G

        def idx_start(g, ib):
            pltpu.async_copy(e3_hbm.at[cbase + g], ebuf[ib], semi[ib])

        def idx_wait(g, ib):
            pltpu.make_async_copy(e3_hbm.at[cbase + g], ebuf[ib],
                                  semi[ib]).wait()

        def gat_start(g, ib, b):
            pltpu.async_copy(xs_hbm.at[ebuf[ib].at[0]], rows[b], semg[b])

        def gat_wait(g, ib, b):
            pltpu.make_async_copy(xs_hbm.at[ebuf[ib].at[0]], rows[b],
                                  semg[b]).wait()

        for g in range(4):
            idx_start(g, g)

        def zrow(i, carry):
            for j in range(DW):
                rows1[i, pl.ds(j * 16, 16)] = z16
            return carry

        lax.fori_loop(0, RC, zrow, 0)
        idx_wait(0, 0)
        gat_start(0, 0, 0)
        for t in range(5):
            pltpu.sync_copy(rows1.at[pl.ds(0, RC)],
                            acc.at[pl.ds(s * RPS + t * RC, RC)])
        plsc.subcore_barrier()
        idx_wait(1, 1)
        gat_start(1, 1, 1)

        def body(t, carry):
            for j in range(4):
                g = 4 * t + j
                b = j % 2
                ib = j % 4
                gat_wait(g, ib, b)
                pltpu.sync_copy(rows[b], acc.at[ebuf[ib].at[1]], add=True)

                @pl.when(g + 4 < G)
                def _():
                    idx_start(g + 4, ib)

                idx_wait(g + 2, (j + 2) % 4)
                gat_start(g + 2, (j + 2) % 4, b)
            return carry

        lax.fori_loop(0, GS // 4, body, 0)

        for g in range(GS, G):
            j = g % 4
            b = j % 2
            ib = j % 4
            gat_wait(g, ib, b)
            pltpu.sync_copy(rows[b], acc.at[ebuf[ib].at[1]], add=True)
            if g + 2 < G:
                idx_wait(g + 2, (j + 2) % 4)
                gat_start(g + 2, (j + 2) % 4, b)

        if T > 0:
            pltpu.sync_copy(tails_hbm.at[wid, 0], tsrc)
            pltpu.sync_copy(tails_hbm.at[wid, 1], tdst)
            pltpu.async_copy(xs_hbm.at[tsrc], rows0.at[pl.ds(0, T)],
                             semg0).wait()
            pltpu.sync_copy(rows0.at[pl.ds(0, T)], acc.at[tdst], add=True)
        plsc.subcore_barrier()
        for t in range(5):
            r0 = s * RPS + t * RC
            pltpu.sync_copy(acc.at[pl.ds(r0, RC)], out_hbm.at[c, pl.ds(r0, RC)])

    return edge_kernel


def _tc_prescale(x, dp0, dp1):
    N, Din = x.shape

    def body(x_ref, dp0_ref, dp1_ref, xs_ref, dis_ref):
        d = 1.0 + dp0_ref[:N, :] + dp1_ref[:N, :]
        dis = lax.rsqrt(d)
        dis_ref[...] = dis
        xs_ref[...] = x_ref[...] * dis

    return pl.pallas_call(
        body,
        out_shape=(jax.ShapeDtypeStruct((N, Din), jnp.float32),
                   jax.ShapeDtypeStruct((N, 1), jnp.float32)),
    )(x, dp0, dp1)


def _tc_layer(segp, xs, dis, W, b, g, be, W_next):
    N, D = xs.shape
    Dout = W_next.shape[1] if W_next is not None else W.shape[1]

    def body(*refs):
        if W_next is None:
            segp_ref, xs_ref, dis_ref, W_ref, b_ref, g_ref, be_ref, out_ref = refs
        else:
            segp_ref, xs_ref, dis_ref, W_ref, b_ref, g_ref, be_ref, Wn_ref, out_ref = refs
        dis_v = dis_ref[...]
        agg = (segp_ref[0, :N, :] + segp_ref[1, :N, :] + xs_ref[...]) * dis_v
        h = jnp.dot(agg, W_ref[...], preferred_element_type=jnp.float32) + b_ref[...]
        mu = jnp.mean(h, axis=0, keepdims=True)
        cen = h - mu
        var = jnp.mean(cen * cen, axis=0, keepdims=True)
        r = jnp.maximum(cen * lax.rsqrt(var + 1e-5) * g_ref[...] + be_ref[...], 0.0)
        if W_next is not None:
            r = jnp.dot(r, Wn_ref[...], preferred_element_type=jnp.float32)
        out_ref[...] = r * dis_v

    args = (segp, xs, dis, W, b, g, be) + (() if W_next is None else (W_next,))
    return pl.pallas_call(
        body,
        out_shape=jax.ShapeDtypeStruct((N, Dout), jnp.float32),
    )(*args)


def _tc_final(segp, zs, dis, b3p, D_OUT):
    N, Dp = zs.shape

    def body(segp_ref, zs_ref, dis_ref, b3_ref, out_ref):
        agg = (segp_ref[0, :N, :] + segp_ref[1, :N, :] + zs_ref[...]) * dis_ref[...] \
              + b3_ref[...]
        col = lax.broadcasted_iota(jnp.int32, agg.shape, 1)
        valid = col < D_OUT
        hm = jnp.where(valid, agg, -1e30)
        m = jnp.max(hm, axis=-1, keepdims=True)
        e = jnp.where(valid, jnp.exp(agg - m), 0.0)
        lse = jnp.log(jnp.sum(e, axis=-1, keepdims=True)) + m
        out_ref[...] = (agg - lse)[:, :D_OUT]

    return pl.pallas_call(
        body,
        out_shape=jax.ShapeDtypeStruct((N, D_OUT), jnp.float32),
    )(segp, zs, dis, b3p)


def kernel(x, edge_index, W1, b1, g1, be1, W2, b2, g2, be2, W3, b3):
    N, D = x.shape
    E = edge_index.shape[1]
    D_OUT = W3.shape[1]
    DP = 64
    NP = 10240

    src = edge_index[0]
    dst = edge_index[1]

    K = 128
    EW = E // NWORK
    G = EW // K
    T = EW - G * K
    ei_w = edge_index.reshape(2, NWORK, EW)
    main = ei_w[:, :, :G * K].reshape(2, NWORK * G, K).transpose(1, 0, 2)
    tails = ei_w[:, :, G * K:].transpose(1, 0, 2)
    K3 = 256
    G3 = EW // K3
    main3 = ei_w[:, :, :G3 * K3].reshape(2, NWORK * G3, K3).transpose(1, 0, 2)

    deg_call = _make_deg(NP, E)
    edge128 = _make_edge(NP, E, D, K, tc_tiling=False)
    edge64 = _make_edge(NP, E, DP, K3, tc_tiling=False)

    dp0, dp1 = deg_call(dst)
    xs1, dis = _tc_prescale(x, dp0.reshape(NP, 1), dp1.reshape(NP, 1))
    seg1 = edge128(main, tails, xs1)
    xs2 = _tc_layer(seg1, xs1, dis, W1, b1, g1, be1, None)
    seg2 = edge128(main, tails, xs2)
    W3p = jnp.pad(W3, ((0, 0), (0, DP - D_OUT)))
    zs = _tc_layer(seg2, xs2, dis, W2, b2, g2, be2, W3p)
    seg3 = edge64(main3, tails, zs)
    b3p = jnp.pad(b3, (0, DP - D_OUT))
    return _tc_final(seg3, zs, dis, b3p, D_OUT)


# speedup vs baseline: 1.0023x; 1.0023x over previous
import functools

import jax
import jax.numpy as jnp
from jax import lax
from jax.experimental import pallas as pl
from jax.experimental.pallas import tpu as pltpu
from jax.experimental.pallas import tpu_sc as plsc

NCORE = 2
NSUB = 16
NWORK = NCORE * NSUB


def _mesh():
    return plsc.VectorSubcoreMesh(core_axis_name="c", subcore_axis_name="s")


def _make_deg(NP, E):
    EW = E // NWORK
    K = 2000
    G = EW // K
    RPS = NP // NSUB

    @functools.partial(
        pl.kernel,
        out_type=(jax.ShapeDtypeStruct((NP,), jnp.float32),
                  jax.ShapeDtypeStruct((NP,), jnp.float32)),
        mesh=_mesh(),
        scratch_types=[
            pltpu.VMEM((K,), jnp.int32),
            pltpu.VMEM((K,), jnp.float32),
            pltpu.VMEM_SHARED((NP,), jnp.float32),
        ],
    )
    def deg_kernel(dst_hbm, out0_hbm, out1_hbm, idx_v, ones_v, acc):
        c = lax.axis_index("c")
        s = lax.axis_index("s")
        wid = c * NSUB + s
        one16 = jnp.ones((16,), jnp.float32)
        z16 = jnp.zeros((16,), jnp.float32)

        def fill(i, carry):
            ones_v[pl.ds(i * 16, 16)] = jnp.where(i * 16 < RPS, z16, one16)
            return carry

        lax.fori_loop(0, K // 16, fill, 0)
        pltpu.sync_copy(ones_v.at[pl.ds(0, RPS)], acc.at[pl.ds(s * RPS, RPS)])
        plsc.subcore_barrier()

        def refill(i, carry):
            ones_v[pl.ds(i * 16, 16)] = one16
            return carry

        lax.fori_loop(0, RPS // 16, refill, 0)

        base = wid * EW

        def body(g, carry):
            pltpu.sync_copy(dst_hbm.at[pl.ds(base + g * K, K)], idx_v)
            pltpu.sync_copy(ones_v, acc.at[idx_v], add=True)
            return carry

        lax.fori_loop(0, G, body, 0)
        plsc.subcore_barrier()

        @pl.when(c == 0)
        def _():
            pltpu.sync_copy(acc.at[pl.ds(s * RPS, RPS)],
                            out0_hbm.at[pl.ds(s * RPS, RPS)])

        @pl.when(c == 1)
        def _():
            pltpu.sync_copy(acc.at[pl.ds(s * RPS, RPS)],
                            out1_hbm.at[pl.ds(s * RPS, RPS)])

    return deg_kernel


def _make_edge(NP, E, D, K, tc_tiling=True):
    EW = E // NWORK
    G = EW // K
    T = EW - G * K
    GS = ((G - 2) // 4) * 4
    RPS = NP // NSUB
    RC = RPS // 5
    DW = D // 16

    @functools.partial(
        pl.kernel,
        out_type=jax.ShapeDtypeStruct((NCORE, NP, D), jnp.float32),
        mesh=_mesh(),
        compiler_params=None if tc_tiling else pltpu.CompilerParams(
            use_tc_tiling_on_sc=False),
        scratch_types=[
            pltpu.VMEM((2, K), jnp.int32),
            pltpu.VMEM((2, K), jnp.int32),
            pltpu.VMEM((2, K), jnp.int32),
            pltpu.VMEM((2, K), jnp.int32),
            pltpu.VMEM((K, D), jnp.float32),
            pltpu.VMEM((K, D), jnp.float32),
            pltpu.SemaphoreType.DMA,
            pltpu.SemaphoreType.DMA,
            pltpu.SemaphoreType.DMA,
            pltpu.SemaphoreType.DMA,
            pltpu.SemaphoreType.DMA,
            pltpu.SemaphoreType.DMA,
            pltpu.VMEM((T if T > 0 else 16,), jnp.int32),
            pltpu.VMEM((T if T > 0 else 16,), jnp.int32),
            pltpu.VMEM_SHARED((NP, D), jnp.float32),
        ],
    )
    def edge_kernel(e3_hbm, tails_hbm, xs_hbm, out_hbm, eb0, eb1, eb2, eb3,
                    rows0, rows1, semg0, semg1, semi0, semi1, semi2, semi3,
                    tsrc, tdst, acc):
        c = lax.axis_index("c")
        s = lax.axis_index("s")
        wid = c * NSUB + s
        z16 = jnp.zeros((16,), jnp.float32)
        ebuf = (eb0, eb1, eb2, eb3)
        semi = (semi0, semi1, semi2, semi3)
        rows = (rows0, rows1)
        semg = (semg0, semg1)

        cbase = wid * G

        def idx_start(g, ib):
            pltpu.async_copy(e3_hbm.at[cbase + g], ebuf[ib], semi[ib])

        def idx_wait(g, ib):
            pltpu.make_async_copy(e3_hbm.at[cbase + g], ebuf[ib],
                                  semi[ib]).wait()

        def gat_start(g, ib, b):
            pltpu.async_copy(xs_hbm.at[ebuf[ib].at[0]], rows[b], semg[b])

        def gat_wait(g, ib, b):
            pltpu.make_async_copy(xs_hbm.at[ebuf[ib].at[0]], rows[b],
                                  semg[b]).wait()

        for g in range(4):
            idx_start(g, g)

        def zrow(i, carry):
            for j in range(DW):
                rows1[i, pl.ds(j * 16, 16)] = z16
            return carry

        lax.fori_loop(0, RC, zrow, 0)
        idx_wait(0, 0)
        gat_start(0, 0, 0)
        for t in range(5):
            pltpu.sync_copy(rows1.at[pl.ds(0, RC)],
                            acc.at[pl.ds(s * RPS + t * RC, RC)])
        plsc.subcore_barrier()
        idx_wait(1, 1)
        gat_start(1, 1, 1)

        def body(t, carry):
            for j in range(4):
                g = 4 * t + j
                b = j % 2
                ib = j % 4
                gat_wait(g, ib, b)
                pltpu.sync_copy(rows[b], acc.at[ebuf[ib].at[1]], add=True)

                @pl.when(g + 4 < G)
                def _():
                    idx_start(g + 4, ib)

                idx_wait(g + 2, (j + 2) % 4)
                gat_start(g + 2, (j + 2) % 4, b)
            return carry

        lax.fori_loop(0, GS // 4, body, 0)

        for g in range(GS, G):
            j = g % 4
            b = j % 2
            ib = j % 4
            gat_wait(g, ib, b)
            pltpu.sync_copy(rows[b], acc.at[ebuf[ib].at[1]], add=True)
            if g + 2 < G:
                idx_wait(g + 2, (j + 2) % 4)
                gat_start(g + 2, (j + 2) % 4, b)

        if T > 0:
            pltpu.sync_copy(tails_hbm.at[wid, 0], tsrc)
            pltpu.sync_copy(tails_hbm.at[wid, 1], tdst)
            pltpu.async_copy(xs_hbm.at[tsrc], rows0.at[pl.ds(0, T)],
                             semg0).wait()
            pltpu.sync_copy(rows0.at[pl.ds(0, T)], acc.at[tdst], add=True)
        plsc.subcore_barrier()
        for t in range(5):
            r0 = s * RPS + t * RC
            pltpu.sync_copy(acc.at[pl.ds(r0, RC)], out_hbm.at[c, pl.ds(r0, RC)])

    return edge_kernel


def _tc_prescale(x, dp0, dp1):
    N, Din = x.shape

    def body(x_ref, dp0_ref, dp1_ref, xs_ref, dis_ref):
        d = 1.0 + dp0_ref[:N, :] + dp1_ref[:N, :]
        dis = lax.rsqrt(d)
        dis_ref[...] = dis
        xs_ref[...] = x_ref[...] * dis

    return pl.pallas_call(
        body,
        out_shape=(jax.ShapeDtypeStruct((N, Din), jnp.float32),
                   jax.ShapeDtypeStruct((N, 1), jnp.float32)),
    )(x, dp0, dp1)


def _tc_layer(segp, xs, dis, W, b, g, be, W_next):
    N, D = xs.shape
    Dout = W_next.shape[1] if W_next is not None else W.shape[1]

    def body(*refs):
        if W_next is None:
            segp_ref, xs_ref, dis_ref, W_ref, b_ref, g_ref, be_ref, out_ref = refs
        else:
            segp_ref, xs_ref, dis_ref, W_ref, b_ref, g_ref, be_ref, Wn_ref, out_ref = refs
        dis_v = dis_ref[...]
        agg = (segp_ref[0, :N, :] + segp_ref[1, :N, :] + xs_ref[...]) * dis_v
        h = jnp.dot(agg, W_ref[...], preferred_element_type=jnp.float32) + b_ref[...]
        mu = jnp.mean(h, axis=0, keepdims=True)
        cen = h - mu
        var = jnp.mean(cen * cen, axis=0, keepdims=True)
        r = jnp.maximum(cen * lax.rsqrt(var + 1e-5) * g_ref[...] + be_ref[...], 0.0)
        if W_next is not None:
            r = jnp.dot(r, Wn_ref[...], preferred_element_type=jnp.float32)
        out_ref[...] = r * dis_v

    args = (segp, xs, dis, W, b, g, be) + (() if W_next is None else (W_next,))
    return pl.pallas_call(
        body,
        out_shape=jax.ShapeDtypeStruct((N, Dout), jnp.float32),
    )(*args)


def _tc_final(segp, zs, dis, b3p, D_OUT):
    N, Dp = zs.shape

    def body(segp_ref, zs_ref, dis_ref, b3_ref, out_ref):
        agg = (segp_ref[0, :N, :] + segp_ref[1, :N, :] + zs_ref[...]) * dis_ref[...] \
              + b3_ref[...]
        col = lax.broadcasted_iota(jnp.int32, agg.shape, 1)
        valid = col < D_OUT
        hm = jnp.where(valid, agg, -1e30)
        m = jnp.max(hm, axis=-1, keepdims=True)
        e = jnp.where(valid, jnp.exp(agg - m), 0.0)
        lse = jnp.log(jnp.sum(e, axis=-1, keepdims=True)) + m
        out_ref[...] = (agg - lse)[:, :D_OUT]

    return pl.pallas_call(
        body,
        out_shape=jax.ShapeDtypeStruct((N, D_OUT), jnp.float32),
    )(segp, zs, dis, b3p)


def kernel(x, edge_index, W1, b1, g1, be1, W2, b2, g2, be2, W3, b3):
    N, D = x.shape
    E = edge_index.shape[1]
    D_OUT = W3.shape[1]
    DP = 64
    NP = 10240

    src = edge_index[0]
    dst = edge_index[1]

    K = 128
    EW = E // NWORK
    G = EW // K
    T = EW - G * K
    ei_w = edge_index.reshape(2, NWORK, EW)
    main = ei_w[:, :, :G * K].reshape(2, NWORK * G, K).transpose(1, 0, 2)
    tails = ei_w[:, :, G * K:].transpose(1, 0, 2)
    K3 = 256
    G3 = EW // K3
    main3 = ei_w[:, :, :G3 * K3].reshape(2, NWORK * G3, K3).transpose(1, 0, 2)

    deg_call = _make_deg(NP, E)
    edge128 = _make_edge(NP, E, D, K)
    edge64 = _make_edge(NP, E, DP, K3, tc_tiling=False)

    dp0, dp1 = deg_call(dst)
    xs1, dis = _tc_prescale(x, dp0.reshape(NP, 1), dp1.reshape(NP, 1))
    seg1 = edge128(main, tails, xs1)
    xs2 = _tc_layer(seg1, xs1, dis, W1, b1, g1, be1, None)
    seg2 = edge128(main, tails, xs2)
    W3p = jnp.pad(W3, ((0, 0), (0, DP - D_OUT)))
    zs = _tc_layer(seg2, xs2, dis, W2, b2, g2, be2, W3p)
    seg3 = edge64(main3, tails, zs)
    b3p = jnp.pad(b3, (0, DP - D_OUT))
    return _tc_final(seg3, zs, dis, b3p, D_OUT)
